# Initial kernel scaffold; baseline (speedup 1.0000x reference)
import jax
import jax.numpy as jnp
from jax.experimental import pallas as pl


def kernel(coords, params):
    raise NotImplementedError("write your pallas kernel here")


# speedup vs baseline: 1.5556x; 1.5556x over previous
import functools

import numpy as np
import jax
import jax.numpy as jnp
from jax import lax
from jax.experimental import pallas as pl
from jax.experimental.pallas import tpu as pltpu
from jax.experimental.pallas import tpu_sc as plsc

N_LVL = 16
F = 4
N_PTS = 131072
P1 = np.int32(np.uint32(2654435761 & 0xFFFFFFFF))
P2 = np.int32(805459861)
HASH_MASK = np.int32((1 << 19) - 1)

_LENS = [min(((16 << l) ** 3 + 7) // 8 * 8, 1 << 19) for l in range(N_LVL)]
_OFFS = [0]
for _l in range(1, N_LVL):
    _OFFS.append(_OFFS[-1] + _LENS[_l - 1])
TOTAL_ROWS = _OFFS[-1] + _LENS[-1]

NW = 32
NP = N_PTS // NW
C = 256
NCH = NP // C
NG = C // 16
NIDX = 8 * C


def _tec_body(coords_hbm, table_hbm, out_hbm, cx, cy, cz, idx_buf, rows,
              frac, acc, sem):
    cid = lax.axis_index("c")
    sid = lax.axis_index("s")
    wid = sid * 2 + cid
    base = wid * NP
    pltpu.sync_copy(coords_hbm.at[pl.ds(base, NP)], cx)
    pltpu.sync_copy(coords_hbm.at[pl.ds(N_PTS + base, NP)], cy)
    pltpu.sync_copy(coords_hbm.at[pl.ds(2 * N_PTS + base, NP)], cz)

    iota = lax.iota(jnp.int32, 16)

    def corner_parts(g, ch, scale_f):
        s = ch * C + g * 16
        px = cx[pl.ds(s, 16)] * scale_f + 0.5
        py = cy[pl.ds(s, 16)] * scale_f + 0.5
        pz = cz[pl.ds(s, 16)] * scale_f + 0.5
        ix = px.astype(jnp.int32)
        iy = py.astype(jnp.int32)
        iz = pz.astype(jnp.int32)
        frac[pl.ds(g * 16, 16)] = px - ix.astype(jnp.float32)
        frac[pl.ds(C + g * 16, 16)] = py - iy.astype(jnp.float32)
        frac[pl.ds(2 * C + g * 16, 16)] = pz - iz.astype(jnp.float32)
        return ix, iy, iz

    def store_corner_idx(g, x0, x1, s00, s01, s10, s11, combine, mask, off):
        gh = lax.shift_right_logical(g, 3)
        col = (g & 7) * 16
        parts = (combine(x0, s00), combine(x0, s01), combine(x0, s10),
                 combine(x0, s11), combine(x1, s00), combine(x1, s01),
                 combine(x1, s10), combine(x1, s11))
        for c in range(8):
            idx_buf[c * 2 + gh, pl.ds(col, 16)] = (parts[c] & mask) + off

    def idx_linear(ch, lvl):
        res = 16 << lvl
        mask = jnp.int32(res ** 3 - 1)
        off = jnp.int32(_OFFS[lvl])
        scale_f = jnp.float32(res - 1)

        def grp(g, carry):
            ix, iy, iz = corner_parts(g, ch, scale_f)
            hy0 = iy * res
            hy1 = hy0 + res
            hz0 = iz * (res * res)
            hz1 = hz0 + res * res
            store_corner_idx(g, ix, ix + 1, hy0 + hz0, hy0 + hz1,
                             hy1 + hz0, hy1 + hz1,
                             lambda a, b: a + b, mask, off)
            return carry

        lax.fori_loop(0, NG, grp, 0, unroll=2)

    def idx_hashed(ch, scale_f, off):
        def grp(g, carry):
            ix, iy, iz = corner_parts(g, ch, scale_f)
            hy0 = iy * P1
            hy1 = hy0 + P1
            hz0 = iz * P2
            hz1 = hz0 + P2
            store_corner_idx(g, ix, ix + 1, hy0 ^ hz0, hy0 ^ hz1,
                             hy1 ^ hz0, hy1 ^ hz1,
                             lambda a, b: a ^ b, HASH_MASK, off)
            return carry

        lax.fori_loop(0, NG, grp, 0, unroll=2)

    def accumulate(ch, col0):
        def grp(g, carry):
            o = g * 16
            fx = frac[pl.ds(o, 16)]
            fy = frac[pl.ds(C + o, 16)]
            fz = frac[pl.ds(2 * C + o, 16)]
            gx = 1.0 - fx
            gy = 1.0 - fy
            gz = 1.0 - fz
            a00 = gx * gy
            a01 = gx * fy
            a10 = fx * gy
            a11 = fx * fy
            w = (a00 * gz, a00 * fz, a01 * gz, a01 * fz,
                 a10 * gz, a10 * fz, a11 * gz, a11 * fz)
            rowb = o + iota
            for f in range(F):
                colf = jnp.full((16,), f, jnp.int32)
                v = w[0] * plsc.load_gather(rows, [rowb, colf])
                for c in range(1, 8):
                    v = v + w[c] * plsc.load_gather(rows, [rowb + c * C, colf])
                plsc.store_scatter(
                    acc, [rowb, jnp.broadcast_to(col0 + f, (16,))], v)
            return carry

        lax.fori_loop(0, NG, grp, 0)

    def gather_rows():
        copies = [
            pltpu.async_copy(table_hbm.at[idx_buf.at[j]],
                             rows.at[pl.ds(j * 128, 128), :], sem)
            for j in range(NIDX // 128)
        ]
        for cp in copies:
            cp.wait()

    def do_chunk(ch, carry):
        for lvl in range(3):
            idx_linear(ch, lvl)
            gather_rows()
            accumulate(ch, jnp.int32(lvl * 4))

        def hlvl(l, c2):
            scale_f = lax.shift_left(16, l).astype(jnp.float32) - 1.0
            off = 299008 + (l - 3) * 524288
            idx_hashed(ch, scale_f, off)
            gather_rows()
            accumulate(ch, l * 4)
            return c2

        lax.fori_loop(3, N_LVL, hlvl, 0)
        pltpu.sync_copy(acc, out_hbm.at[pl.ds(base + ch * C, C), :])
        return carry

    lax.fori_loop(0, NCH, do_chunk, 0)


@jax.jit
def kernel(coords, params):
    coords_t = coords.T.reshape(3 * N_PTS)
    table = params.reshape(TOTAL_ROWS, F)
    mesh = plsc.VectorSubcoreMesh(core_axis_name="c", subcore_axis_name="s")
    run = pl.kernel(
        _tec_body,
        out_type=jax.ShapeDtypeStruct((N_PTS, N_LVL * F), jnp.float32),
        mesh=mesh,
        scratch_types=[
            pltpu.VMEM((NP,), jnp.float32),
            pltpu.VMEM((NP,), jnp.float32),
            pltpu.VMEM((NP,), jnp.float32),
            pltpu.VMEM((NIDX // 128, 128), jnp.int32),
            pltpu.VMEM((NIDX, F), jnp.float32),
            pltpu.VMEM((3 * C,), jnp.float32),
            pltpu.VMEM((C, N_LVL * F), jnp.float32),
            pltpu.SemaphoreType.DMA,
        ],
        compiler_params=pltpu.CompilerParams(
            needs_layout_passes=False, use_tc_tiling_on_sc=False),
    )
    return run(coords_t, table)


# speedup vs baseline: 9.0755x; 5.8340x over previous
import functools

import numpy as np
import jax
import jax.numpy as jnp
from jax import lax
from jax.experimental import pallas as pl
from jax.experimental.pallas import tpu as pltpu
from jax.experimental.pallas import tpu_sc as plsc

N_LVL = 16
F = 4
N_PTS = 131072
P1 = np.int32(np.uint32(2654435761 & 0xFFFFFFFF))
P2 = np.int32(805459861)
HASH_MASK = np.int32((1 << 19) - 1)

_LENS = [min(((16 << l) ** 3 + 7) // 8 * 8, 1 << 19) for l in range(N_LVL)]
_OFFS = [0]
for _l in range(1, N_LVL):
    _OFFS.append(_OFFS[-1] + _LENS[_l - 1])
TOTAL_ROWS = _OFFS[-1] + _LENS[-1]

NW = 32
NP = N_PTS // NW
C = 256
NCH = NP // C
NG = C // 16
NIDX = 8 * C


def _tec_body(coords_hbm, table_hbm, out_hbm, cx, cy, cz, idx_buf,
              rows, acc, sem):
    cid = lax.axis_index("c")
    sid = lax.axis_index("s")
    wid = sid * 2 + cid
    base = wid * NP
    pltpu.sync_copy(coords_hbm.at[pl.ds(base, NP)], cx)
    pltpu.sync_copy(coords_hbm.at[pl.ds(N_PTS + base, NP)], cy)
    pltpu.sync_copy(coords_hbm.at[pl.ds(2 * N_PTS + base, NP)], cz)

    iota = lax.iota(jnp.int32, 16)

    def positions(g, ch, scale_f):
        s = ch * C + g * 16
        px = cx[pl.ds(s, 16)] * scale_f + 0.5
        py = cy[pl.ds(s, 16)] * scale_f + 0.5
        pz = cz[pl.ds(s, 16)] * scale_f + 0.5
        ix = px.astype(jnp.int32)
        iy = py.astype(jnp.int32)
        iz = pz.astype(jnp.int32)
        return px, py, pz, ix, iy, iz

    def store_corner_idx(g, x0, x1, s00, s01, s10, s11, combine, mask):
        gh = lax.shift_right_logical(g, 3)
        col = (g & 7) * 16
        parts = (combine(x0, s00), combine(x0, s01), combine(x0, s10),
                 combine(x0, s11), combine(x1, s00), combine(x1, s01),
                 combine(x1, s10), combine(x1, s11))
        for c in range(8):
            h = parts[c] & mask
            idx_buf[c * 2 + gh, pl.ds(col, 16)] = lax.shift_right_logical(h, 1)

    def idx_linear(ch, lvl):
        res = 16 << lvl
        mask = jnp.int32(res ** 3 - 1)
        scale_f = jnp.float32(res - 1)

        def grp(g, carry):
            _, _, _, ix, iy, iz = positions(g, ch, scale_f)
            hy0 = iy * res
            hy1 = hy0 + res
            hz0 = iz * (res * res)
            hz1 = hz0 + res * res
            store_corner_idx(g, ix, ix + 1, hy0 + hz0, hy0 + hz1,
                             hy1 + hz0, hy1 + hz1,
                             lambda a, b: a + b, mask)
            return carry

        lax.fori_loop(0, NG, grp, 0, unroll=2)

    def idx_hashed(ch, scale_f):
        def grp(g, carry):
            _, _, _, ix, iy, iz = positions(g, ch, scale_f)
            hy0 = iy * P1
            hy1 = hy0 + P1
            hz0 = iz * P2
            hz1 = hz0 + P2
            store_corner_idx(g, ix, ix + 1, hy0 ^ hz0, hy0 ^ hz1,
                             hy1 ^ hz0, hy1 ^ hz1,
                             lambda a, b: a ^ b, HASH_MASK)
            return carry

        lax.fori_loop(0, NG, grp, 0, unroll=2)

    def accumulate(ch, col0, scale_f, xor_all):
        ecs = (0, 1, 1, 0, 1, 0, 0, 1) if xor_all else (0, 0, 0, 0, 1, 1, 1, 1)

        def grp(g, carry):
            o = g * 16
            px, py, pz, ix, iy, iz = positions(g, ch, scale_f)
            fx = px - ix.astype(jnp.float32)
            fy = py - iy.astype(jnp.float32)
            fz = pz - iz.astype(jnp.float32)
            gx = 1.0 - fx
            gy = 1.0 - fy
            gz = 1.0 - fz
            a00 = gx * gy
            a01 = gx * fy
            a10 = fx * gy
            a11 = fx * fy
            w = (a00 * gz, a00 * fz, a01 * gz, a01 * fz,
                 a10 * gz, a10 * fz, a11 * gz, a11 * fz)
            par = (ix ^ iy ^ iz) if xor_all else ix
            pb = (par & 1).astype(jnp.float32)
            qb = 1.0 - pb
            wb = [w[c] * (qb if ecs[c] else pb) for c in range(8)]
            wa = [w[c] - wb[c] for c in range(8)]
            rowb = o + iota
            for f in range(F):
                ca = jnp.full((16,), f, jnp.int32)
                cb = jnp.full((16,), F + f, jnp.int32)
                v = wa[0] * plsc.load_gather(rows, [rowb, ca])
                v = v + wb[0] * plsc.load_gather(rows, [rowb, cb])
                for c in range(1, 8):
                    rc = rowb + c * C
                    v = v + wa[c] * plsc.load_gather(rows, [rc, ca])
                    v = v + wb[c] * plsc.load_gather(rows, [rc, cb])
                plsc.store_scatter(
                    acc, [rowb, jnp.broadcast_to(col0 + f, (16,))], v)
            return carry

        lax.fori_loop(0, NG, grp, 0)

    def gather_rows():
        copies = [
            pltpu.async_copy(table_hbm.at[idx_buf.at[j]],
                             rows.at[pl.ds(j * 128, 128), :], sem)
            for j in range(NIDX // 128)
        ]
        for cp in copies:
            cp.wait()

    def do_chunk(ch, carry):
        for lvl in range(3):
            idx_linear(ch, lvl)
            gather_rows()
            accumulate(ch, jnp.int32(lvl * 4), jnp.float32((16 << lvl) - 1),
                       xor_all=False)

        def hlvl(l, c2):
            scale_f = lax.shift_left(16, l).astype(jnp.float32) - 1.0
            idx_hashed(ch, scale_f)
            gather_rows()
            accumulate(ch, l * 4, scale_f, xor_all=True)
            return c2

        lax.fori_loop(3, N_LVL, hlvl, 0)
        pltpu.sync_copy(acc, out_hbm.at[pl.ds(base + ch * C, C), :])
        return carry

    lax.fori_loop(0, NCH, do_chunk, 0)


@jax.jit
def kernel(coords, params):
    coords_t = coords.T.reshape(3 * N_PTS)
    table = params.reshape(TOTAL_ROWS // 2, 2 * F)
    mesh = plsc.VectorSubcoreMesh(core_axis_name="c", subcore_axis_name="s")
    run = pl.kernel(
        _tec_body,
        out_type=jax.ShapeDtypeStruct((N_PTS, N_LVL * F), jnp.float32),
        mesh=mesh,
        scratch_types=[
            pltpu.VMEM((NP,), jnp.float32),
            pltpu.VMEM((NP,), jnp.float32),
            pltpu.VMEM((NP,), jnp.float32),
            pltpu.VMEM((NIDX // 128, 128), jnp.int32),
            pltpu.VMEM((NIDX, 2 * F), jnp.float32),
            pltpu.VMEM((C, N_LVL * F), jnp.float32),
            pltpu.SemaphoreType.DMA,
        ],
        compiler_params=pltpu.CompilerParams(
            needs_layout_passes=False, use_tc_tiling_on_sc=False),
    )
    return run(coords_t, table)


# speedup vs baseline: 13.6995x; 1.5095x over previous
import functools

import numpy as np
import jax
import jax.numpy as jnp
from jax import lax
from jax.experimental import pallas as pl
from jax.experimental.pallas import tpu as pltpu
from jax.experimental.pallas import tpu_sc as plsc

N_LVL = 16
F = 4
N_PTS = 131072
P1 = np.int32(np.uint32(2654435761 & 0xFFFFFFFF))
P2 = np.int32(805459861)
HASH_MASK = np.int32((1 << 19) - 1)

_LENS = [min(((16 << l) ** 3 + 7) // 8 * 8, 1 << 19) for l in range(N_LVL)]
_OFFS = [0]
for _l in range(1, N_LVL):
    _OFFS.append(_OFFS[-1] + _LENS[_l - 1])
TOTAL_ROWS = _OFFS[-1] + _LENS[-1]

NW = 32
NP = N_PTS // NW
C = 256
NCH = NP // C
NG = C // 16
NIDX = 8 * C


def _tec_body(coords_hbm, table_hbm, out_hbm, cx, cy, cz, idx_a, idx_b,
              rows_a, rows_b, acc, sem_a, sem_b):
    cid = lax.axis_index("c")
    sid = lax.axis_index("s")
    wid = sid * 2 + cid
    base = wid * NP
    pltpu.sync_copy(coords_hbm.at[pl.ds(base, NP)], cx)
    pltpu.sync_copy(coords_hbm.at[pl.ds(N_PTS + base, NP)], cy)
    pltpu.sync_copy(coords_hbm.at[pl.ds(2 * N_PTS + base, NP)], cz)

    iota = lax.iota(jnp.int32, 16)

    def positions(g, ch, scale_f):
        s = ch * C + g * 16
        px = cx[pl.ds(s, 16)] * scale_f + 0.5
        py = cy[pl.ds(s, 16)] * scale_f + 0.5
        pz = cz[pl.ds(s, 16)] * scale_f + 0.5
        ix = px.astype(jnp.int32)
        iy = py.astype(jnp.int32)
        iz = pz.astype(jnp.int32)
        return px, py, pz, ix, iy, iz

    def store_corner_idx(idx_buf, g, x0, x1, s00, s01, s10, s11, combine,
                         mask):
        gh = lax.shift_right_logical(g, 3)
        col = (g & 7) * 16
        parts = (combine(x0, s00), combine(x0, s01), combine(x0, s10),
                 combine(x0, s11), combine(x1, s00), combine(x1, s01),
                 combine(x1, s10), combine(x1, s11))
        for c in range(8):
            h = parts[c] & mask
            idx_buf[c * 2 + gh, pl.ds(col, 16)] = lax.shift_right_logical(h, 1)

    def idx_linear(idx_buf, ch, lvl):
        res = 16 << lvl
        mask = jnp.int32(res ** 3 - 1)
        scale_f = jnp.float32(res - 1)

        def grp(g, carry):
            _, _, _, ix, iy, iz = positions(g, ch, scale_f)
            hy0 = iy * res
            hy1 = hy0 + res
            hz0 = iz * (res * res)
            hz1 = hz0 + res * res
            store_corner_idx(idx_buf, g, ix, ix + 1, hy0 + hz0, hy0 + hz1,
                             hy1 + hz0, hy1 + hz1,
                             lambda a, b: a + b, mask)
            return carry

        lax.fori_loop(0, NG, grp, 0, unroll=2)

    def idx_hashed(idx_buf, ch, scale_f):
        def grp(g, carry):
            _, _, _, ix, iy, iz = positions(g, ch, scale_f)
            hy0 = iy * P1
            hy1 = hy0 + P1
            hz0 = iz * P2
            hz1 = hz0 + P2
            store_corner_idx(idx_buf, g, ix, ix + 1, hy0 ^ hz0, hy0 ^ hz1,
                             hy1 ^ hz0, hy1 ^ hz1,
                             lambda a, b: a ^ b, HASH_MASK)
            return carry

        lax.fori_loop(0, NG, grp, 0, unroll=2)

    def accumulate(rows, ch, col0, scale_f, xor_all):
        ecs = (0, 1, 1, 0, 1, 0, 0, 1) if xor_all else (0, 0, 0, 0, 1, 1, 1, 1)

        def grp(g, carry):
            o = g * 16
            px, py, pz, ix, iy, iz = positions(g, ch, scale_f)
            fx = px - ix.astype(jnp.float32)
            fy = py - iy.astype(jnp.float32)
            fz = pz - iz.astype(jnp.float32)
            gx = 1.0 - fx
            gy = 1.0 - fy
            gz = 1.0 - fz
            a00 = gx * gy
            a01 = gx * fy
            a10 = fx * gy
            a11 = fx * fy
            w = (a00 * gz, a00 * fz, a01 * gz, a01 * fz,
                 a10 * gz, a10 * fz, a11 * gz, a11 * fz)
            par = (ix ^ iy ^ iz) if xor_all else ix
            pb = (par & 1).astype(jnp.float32)
            qb = 1.0 - pb
            wb = [w[c] * (qb if ecs[c] else pb) for c in range(8)]
            wa = [w[c] - wb[c] for c in range(8)]
            rowb = o + iota
            for f in range(F):
                ca = jnp.full((16,), f, jnp.int32)
                cb = jnp.full((16,), F + f, jnp.int32)
                v = wa[0] * plsc.load_gather(rows, [rowb, ca])
                v = v + wb[0] * plsc.load_gather(rows, [rowb, cb])
                for c in range(1, 8):
                    rc = rowb + c * C
                    v = v + wa[c] * plsc.load_gather(rows, [rc, ca])
                    v = v + wb[c] * plsc.load_gather(rows, [rc, cb])
                plsc.store_scatter(
                    acc, [rowb, jnp.broadcast_to(col0 + f, (16,))], v)
            return carry

        lax.fori_loop(0, NG, grp, 0)

    def descs(idx_buf, rows, sem):
        return [
            pltpu.make_async_copy(table_hbm.at[idx_buf.at[j]],
                                  rows.at[pl.ds(j * 128, 128), :], sem)
            for j in range(NIDX // 128)
        ]

    def fire(idx_buf, rows, sem):
        for d in descs(idx_buf, rows, sem):
            d.start()

    def drain(idx_buf, rows, sem):
        for d in descs(idx_buf, rows, sem):
            d.wait()

    def hscale(l):
        return lax.shift_left(16, l).astype(jnp.float32) - 1.0

    def do_chunk(ch, carry):
        for lvl in range(3):
            idx_linear(idx_a, ch, lvl)
            fire(idx_a, rows_a, sem_a)
            drain(idx_a, rows_a, sem_a)
            accumulate(rows_a, ch, jnp.int32(lvl * 4),
                       jnp.float32((16 << lvl) - 1), xor_all=False)

        idx_hashed(idx_a, ch, hscale(jnp.int32(3)))
        fire(idx_a, rows_a, sem_a)

        def pair(k, c2):
            l0 = 4 + 2 * k
            idx_hashed(idx_b, ch, hscale(l0))
            fire(idx_b, rows_b, sem_b)
            drain(idx_a, rows_a, sem_a)
            accumulate(rows_a, ch, (l0 - 1) * 4, hscale(l0 - 1), xor_all=True)
            idx_hashed(idx_a, ch, hscale(l0 + 1))
            fire(idx_a, rows_a, sem_a)
            drain(idx_b, rows_b, sem_b)
            accumulate(rows_b, ch, l0 * 4, hscale(l0), xor_all=True)
            return c2

        lax.fori_loop(0, 6, pair, 0)
        drain(idx_a, rows_a, sem_a)
        accumulate(rows_a, ch, jnp.int32(15 * 4), hscale(jnp.int32(15)),
                   xor_all=True)
        pltpu.sync_copy(acc, out_hbm.at[pl.ds(base + ch * C, C), :])
        return carry

    lax.fori_loop(0, NCH, do_chunk, 0)


@jax.jit
def kernel(coords, params):
    coords_t = coords.T.reshape(3 * N_PTS)
    table = params.reshape(TOTAL_ROWS // 2, 2 * F)
    mesh = plsc.VectorSubcoreMesh(core_axis_name="c", subcore_axis_name="s")
    run = pl.kernel(
        _tec_body,
        out_type=jax.ShapeDtypeStruct((N_PTS, N_LVL * F), jnp.float32),
        mesh=mesh,
        scratch_types=[
            pltpu.VMEM((NP,), jnp.float32),
            pltpu.VMEM((NP,), jnp.float32),
            pltpu.VMEM((NP,), jnp.float32),
            pltpu.VMEM((NIDX // 128, 128), jnp.int32),
            pltpu.VMEM((NIDX // 128, 128), jnp.int32),
            pltpu.VMEM((NIDX, 2 * F), jnp.float32),
            pltpu.VMEM((NIDX, 2 * F), jnp.float32),
            pltpu.VMEM((C, N_LVL * F), jnp.float32),
            pltpu.SemaphoreType.DMA,
            pltpu.SemaphoreType.DMA,
        ],
        compiler_params=pltpu.CompilerParams(
            needs_layout_passes=False, use_tc_tiling_on_sc=False),
    )
    return run(coords_t, table)


# speedup vs baseline: 16.7683x; 1.2240x over previous
import functools

import numpy as np
import jax
import jax.numpy as jnp
from jax import lax
from jax.experimental import pallas as pl
from jax.experimental.pallas import tpu as pltpu
from jax.experimental.pallas import tpu_sc as plsc

N_LVL = 16
F = 4
N_PTS = 131072
P1 = np.int32(np.uint32(2654435761 & 0xFFFFFFFF))
P2 = np.int32(805459861)
HASH_MASK = np.int32((1 << 19) - 1)

_LENS = [min(((16 << l) ** 3 + 7) // 8 * 8, 1 << 19) for l in range(N_LVL)]
_OFFS = [0]
for _l in range(1, N_LVL):
    _OFFS.append(_OFFS[-1] + _LENS[_l - 1])
TOTAL_ROWS = _OFFS[-1] + _LENS[-1]

NW = 32
NP = N_PTS // NW
C = 256
NCH = NP // C
NG = C // 16
NIDX = 8 * C


SPM_ROWS = 131072


def _tec_body(coords_hbm, table_hbm, out_hbm, cx, cy, cz, idx_a, idx_b,
              rows_a, rows_b, acc, shared, sem_a, sem_b):
    cid = lax.axis_index("c")
    sid = lax.axis_index("s")
    wid = sid * 2 + cid
    base = wid * NP
    sl = SPM_ROWS // 16
    pltpu.sync_copy(table_hbm.at[pl.ds(sid * sl, sl), :],
                    shared.at[pl.ds(sid * sl, sl), :])
    pltpu.sync_copy(coords_hbm.at[pl.ds(base, NP)], cx)
    pltpu.sync_copy(coords_hbm.at[pl.ds(N_PTS + base, NP)], cy)
    pltpu.sync_copy(coords_hbm.at[pl.ds(2 * N_PTS + base, NP)], cz)
    plsc.subcore_barrier()

    iota = lax.iota(jnp.int32, 16)

    def positions(g, ch, scale_f):
        s = ch * C + g * 16
        px = cx[pl.ds(s, 16)] * scale_f + 0.5
        py = cy[pl.ds(s, 16)] * scale_f + 0.5
        pz = cz[pl.ds(s, 16)] * scale_f + 0.5
        ix = px.astype(jnp.int32)
        iy = py.astype(jnp.int32)
        iz = pz.astype(jnp.int32)
        return px, py, pz, ix, iy, iz

    def store_corner_idx(idx_buf, g, x0, x1, s00, s01, s10, s11, combine,
                         mask):
        gh = lax.shift_right_logical(g, 3)
        col = (g & 7) * 16
        parts = (combine(x0, s00), combine(x0, s01), combine(x0, s10),
                 combine(x0, s11), combine(x1, s00), combine(x1, s01),
                 combine(x1, s10), combine(x1, s11))
        for c in range(8):
            h = parts[c] & mask
            idx_buf[c * 2 + gh, pl.ds(col, 16)] = lax.shift_right_logical(h, 1)

    def idx_linear(idx_buf, ch, lvl):
        res = 16 << lvl
        mask = jnp.int32(res ** 3 - 1)
        scale_f = jnp.float32(res - 1)

        def grp(g, carry):
            _, _, _, ix, iy, iz = positions(g, ch, scale_f)
            hy0 = iy * res
            hy1 = hy0 + res
            hz0 = iz * (res * res)
            hz1 = hz0 + res * res
            store_corner_idx(idx_buf, g, ix, ix + 1, hy0 + hz0, hy0 + hz1,
                             hy1 + hz0, hy1 + hz1,
                             lambda a, b: a + b, mask)
            return carry

        lax.fori_loop(0, NG, grp, 0, unroll=2)

    def idx_hashed(idx_buf, ch, scale_f):
        def grp(g, carry):
            _, _, _, ix, iy, iz = positions(g, ch, scale_f)
            hy0 = iy * P1
            hy1 = hy0 + P1
            hz0 = iz * P2
            hz1 = hz0 + P2
            store_corner_idx(idx_buf, g, ix, ix + 1, hy0 ^ hz0, hy0 ^ hz1,
                             hy1 ^ hz0, hy1 ^ hz1,
                             lambda a, b: a ^ b, HASH_MASK)
            return carry

        lax.fori_loop(0, NG, grp, 0, unroll=2)

    def accumulate(rows, ch, col0, scale_f, xor_all):
        ecs = (0, 1, 1, 0, 1, 0, 0, 1) if xor_all else (0, 0, 0, 0, 1, 1, 1, 1)

        def grp(g, carry):
            o = g * 16
            px, py, pz, ix, iy, iz = positions(g, ch, scale_f)
            fx = px - ix.astype(jnp.float32)
            fy = py - iy.astype(jnp.float32)
            fz = pz - iz.astype(jnp.float32)
            gx = 1.0 - fx
            gy = 1.0 - fy
            gz = 1.0 - fz
            a00 = gx * gy
            a01 = gx * fy
            a10 = fx * gy
            a11 = fx * fy
            w = (a00 * gz, a00 * fz, a01 * gz, a01 * fz,
                 a10 * gz, a10 * fz, a11 * gz, a11 * fz)
            par = (ix ^ iy ^ iz) if xor_all else ix
            pb = (par & 1).astype(jnp.float32)
            qb = 1.0 - pb
            wb = [w[c] * (qb if ecs[c] else pb) for c in range(8)]
            wa = [w[c] - wb[c] for c in range(8)]
            rowb = o + iota
            for f in range(F):
                ca = jnp.full((16,), f, jnp.int32)
                cb = jnp.full((16,), F + f, jnp.int32)
                v = wa[0] * plsc.load_gather(rows, [rowb, ca])
                v = v + wb[0] * plsc.load_gather(rows, [rowb, cb])
                for c in range(1, 8):
                    rc = rowb + c * C
                    v = v + wa[c] * plsc.load_gather(rows, [rc, ca])
                    v = v + wb[c] * plsc.load_gather(rows, [rc, cb])
                plsc.store_scatter(
                    acc, [rowb, jnp.broadcast_to(col0 + f, (16,))], v)
            return carry

        lax.fori_loop(0, NG, grp, 0)

    def fire(idx_buf, rows, sem, src):
        for j in range(NIDX // 128):
            pltpu.async_copy(src.at[idx_buf.at[j]],
                             rows.at[pl.ds(j * 128, 128), :], sem)

    def drain(idx_buf, rows, sem):
        for j in range(NIDX // 128):
            pltpu.make_async_copy(table_hbm.at[idx_buf.at[j]],
                                  rows.at[pl.ds(j * 128, 128), :], sem).wait()

    def hscale(l):
        return lax.shift_left(16, l).astype(jnp.float32) - 1.0

    def do_chunk(ch, carry):
        idx_linear(idx_a, ch, 0)
        fire(idx_a, rows_a, sem_a, shared)
        idx_linear(idx_b, ch, 1)
        fire(idx_b, rows_b, sem_b, shared)
        drain(idx_a, rows_a, sem_a)
        accumulate(rows_a, ch, jnp.int32(0), jnp.float32(15), xor_all=False)
        idx_linear(idx_a, ch, 2)
        fire(idx_a, rows_a, sem_a, shared)
        drain(idx_b, rows_b, sem_b)
        accumulate(rows_b, ch, jnp.int32(4), jnp.float32(31), xor_all=False)
        idx_hashed(idx_b, ch, hscale(jnp.int32(3)))
        fire(idx_b, rows_b, sem_b, table_hbm)
        drain(idx_a, rows_a, sem_a)
        accumulate(rows_a, ch, jnp.int32(8), jnp.float32(63), xor_all=False)

        def pair(k, c2):
            l0 = 4 + 2 * k
            idx_hashed(idx_a, ch, hscale(l0))
            fire(idx_a, rows_a, sem_a, table_hbm)
            drain(idx_b, rows_b, sem_b)
            accumulate(rows_b, ch, (l0 - 1) * 4, hscale(l0 - 1), xor_all=True)
            idx_hashed(idx_b, ch, hscale(l0 + 1))
            fire(idx_b, rows_b, sem_b, table_hbm)
            drain(idx_a, rows_a, sem_a)
            accumulate(rows_a, ch, l0 * 4, hscale(l0), xor_all=True)
            return c2

        lax.fori_loop(0, 6, pair, 0)
        drain(idx_b, rows_b, sem_b)
        accumulate(rows_b, ch, jnp.int32(15 * 4), hscale(jnp.int32(15)),
                   xor_all=True)
        pltpu.sync_copy(acc, out_hbm.at[pl.ds(base + ch * C, C), :])
        return carry

    lax.fori_loop(0, NCH, do_chunk, 0)


@jax.jit
def kernel(coords, params):
    coords_t = coords.T.reshape(3 * N_PTS)
    table = params.reshape(TOTAL_ROWS // 2, 2 * F)
    mesh = plsc.VectorSubcoreMesh(core_axis_name="c", subcore_axis_name="s")
    run = pl.kernel(
        _tec_body,
        out_type=jax.ShapeDtypeStruct((N_PTS, N_LVL * F), jnp.float32),
        mesh=mesh,
        scratch_types=[
            pltpu.VMEM((NP,), jnp.float32),
            pltpu.VMEM((NP,), jnp.float32),
            pltpu.VMEM((NP,), jnp.float32),
            pltpu.VMEM((NIDX // 128, 128), jnp.int32),
            pltpu.VMEM((NIDX // 128, 128), jnp.int32),
            pltpu.VMEM((NIDX, 2 * F), jnp.float32),
            pltpu.VMEM((NIDX, 2 * F), jnp.float32),
            pltpu.VMEM((C, N_LVL * F), jnp.float32),
            pltpu.VMEM_SHARED((SPM_ROWS, 2 * F), jnp.float32),
            pltpu.SemaphoreType.DMA,
            pltpu.SemaphoreType.DMA,
        ],
        compiler_params=pltpu.CompilerParams(
            needs_layout_passes=False, use_tc_tiling_on_sc=False),
    )
    return run(coords_t, table)
